# baseline (speedup 1.0000x reference)
import functools

import jax
import jax.numpy as jnp
from jax import lax
from jax.experimental import pallas as pl
from jax.experimental.pallas import tpu as pltpu
from jax.experimental.pallas import tpu_sc as plsc

B, C, H, W = 16, 256, 32, 32
K, D = 8192, 256
N = B * H * W
NT = 256
KC = 2048
NKC = K // KC

SC_NC, SC_NS = 2, 16
SC_NW = SC_NC * SC_NS
ROWS_PER_W = N // SC_NW
GCH = 128
NGCH = ROWS_PER_W // GCH


GB = 2736
GROUPS = [(0, GB), (GB, 2 * GB), (2 * GB, K)]


def _dist_argmin_body(x_ref, e_ref, csqr_ref, isqr_ref, idx_ref, perp_ref,
                      counts_ref):
    n = pl.program_id(0)

    @pl.when(n == 0)
    def _init():
        counts_ref[...] = jnp.zeros_like(counts_ref)

    x = x_ref[...]
    xb = x.astype(jnp.bfloat16)
    isqr = isqr_ref[...]
    g_min = [jnp.full((NT, 1), jnp.inf, jnp.float32) for _ in GROUPS]
    g_idx = [jnp.zeros((NT, 1), jnp.int32) for _ in GROUPS]
    for c in range(NKC):
        eb = e_ref[c * KC:(c + 1) * KC, :].astype(jnp.bfloat16)
        mm = lax.dot_general(xb, eb, (((1,), (1,)), ((), ())),
                             preferred_element_type=jnp.float32)
        d = (csqr_ref[:, c * KC:(c + 1) * KC] + isqr) - 2.0 * mm
        ii = lax.broadcasted_iota(jnp.int32, (NT, KC), 1) + (c * KC)
        for g, (g0, g1) in enumerate(GROUPS):
            lo, hi = max(c * KC, g0), min((c + 1) * KC, g1)
            if lo >= hi:
                continue
            if lo == c * KC and hi == (c + 1) * KC:
                dg = d
            else:
                gmask = (ii >= g0) & (ii < g1)
                dg = jnp.where(gmask, d, jnp.inf)
            lm = jnp.min(dg, axis=1, keepdims=True)
            li = jnp.min(jnp.where(dg == lm, ii, K), axis=1, keepdims=True)
            upd = lm < g_min[g]
            g_min[g] = jnp.where(upd, lm, g_min[g])
            g_idx[g] = jnp.where(upd, li, g_idx[g])
    v = g_min[0].astype(jnp.bfloat16).astype(jnp.float32)
    run_idx = g_idx[0]
    for g in range(1, len(GROUPS)):
        upd = g_min[g] < v
        run_idx = jnp.where(upd, g_idx[g], run_idx)
        v = jnp.where(upd, g_min[g], v)
        v = v.astype(jnp.bfloat16).astype(jnp.float32)
    idx_ref[...] = run_idx

    kio = lax.broadcasted_iota(jnp.int32, (NT, K), 1)
    counts_ref[...] += jnp.sum(
        jnp.where(run_idx == kio, 1.0, 0.0), axis=0, keepdims=True)

    @pl.when(n == pl.num_programs(0) - 1)
    def _fin():
        p = counts_ref[...] * (1.0 / N)
        perp_ref[...] = jnp.exp(-jnp.sum(p * jnp.log(p + 1e-10))).reshape(1, 1)


def _dist_argmin(flat_x, emb, csqr, isqr):
    return pl.pallas_call(
        _dist_argmin_body,
        grid=(N // NT,),
        in_specs=[
            pl.BlockSpec((NT, D), lambda n: (n, 0)),
            pl.BlockSpec((K, D), lambda n: (0, 0)),
            pl.BlockSpec((1, K), lambda n: (0, 0)),
            pl.BlockSpec((NT, 1), lambda n: (n, 0)),
        ],
        out_specs=[
            pl.BlockSpec((NT, 1), lambda n: (n, 0)),
            pl.BlockSpec((1, 1), lambda n: (0, 0)),
        ],
        out_shape=[
            jax.ShapeDtypeStruct((N, 1), jnp.int32),
            jax.ShapeDtypeStruct((1, 1), jnp.float32),
        ],
        scratch_shapes=[
            pltpu.VMEM((1, K), jnp.float32),
        ],
        compiler_params=pltpu.CompilerParams(
            dimension_semantics=("arbitrary",)),
    )(flat_x, emb, csqr, isqr)


def _sc_gather(table, idx):
    mesh = plsc.VectorSubcoreMesh(core_axis_name="c", subcore_axis_name="s")

    @functools.partial(
        pl.kernel, mesh=mesh,
        out_type=jax.ShapeDtypeStruct((N, D), jnp.float32),
        scratch_types=[
            pltpu.VMEM((ROWS_PER_W,), jnp.int32),
            pltpu.VMEM((GCH, D), jnp.float32),
            pltpu.VMEM((GCH, D), jnp.float32),
            pltpu.SemaphoreType.DMA,
            pltpu.SemaphoreType.DMA,
        ],
    )
    def gather_kernel(table_hbm, idx_hbm, out_hbm, idx_v, buf0, buf1, s0, s1):
        wid = lax.axis_index("s") * SC_NC + lax.axis_index("c")
        base = wid * ROWS_PER_W
        pltpu.sync_copy(idx_hbm.at[pl.ds(base, ROWS_PER_W)], idx_v)
        bufs, sems = (buf0, buf1), (s0, s1)
        handles = [None, None]
        handles[0] = pltpu.async_copy(
            table_hbm.at[idx_v.at[pl.ds(0, GCH)]], bufs[0], sems[0])
        for c in range(NGCH):
            if c + 1 < NGCH:
                nb = (c + 1) % 2
                handles[nb] = pltpu.async_copy(
                    table_hbm.at[idx_v.at[pl.ds((c + 1) * GCH, GCH)]],
                    bufs[nb], sems[nb])
            b = c % 2
            handles[b].wait()
            pltpu.sync_copy(bufs[b], out_hbm.at[pl.ds(base + c * GCH, GCH)])

    return gather_kernel(table, idx)


def kernel(inputs, embedding_weight):
    x = jnp.transpose(inputs, (0, 2, 3, 1))
    flat_x = x.reshape(N, D)
    csqr = jnp.sum(embedding_weight ** 2, axis=1)[None, :]
    isqr = jnp.sum(flat_x ** 2, axis=1, keepdims=True)
    idx2, perp = _dist_argmin(flat_x, embedding_weight, csqr, isqr)
    indices_out = idx2.reshape(B, H, W)
    q_flat = _sc_gather(embedding_weight, idx2.reshape(N))
    q = q_flat.reshape(B, H, W, D)
    q = x + lax.stop_gradient(q - x)
    quantized = jnp.transpose(q, (0, 3, 1, 2))
    return indices_out, quantized, perp.reshape(())


# speedup vs baseline: 1.1662x; 1.1662x over previous
import functools

import jax
import jax.numpy as jnp
from jax import lax
from jax.experimental import pallas as pl
from jax.experimental.pallas import tpu as pltpu
from jax.experimental.pallas import tpu_sc as plsc

B, C, H, W = 16, 256, 32, 32
K, D = 8192, 256
N = B * H * W
NT = 256
KC = 2048
NKC = K // KC

SC_NC, SC_NS = 2, 16
SC_NW = SC_NC * SC_NS
ROWS_PER_W = N // SC_NW
GCH = 128
NGCH = ROWS_PER_W // GCH


GB = 2736
GROUPS = [(0, GB), (GB, 2 * GB), (2 * GB, K)]


def _dist_argmin_body(x_ref, e_ref, csqr_ref, isqr_ref, idx_ref, perp_ref,
                      counts_ref):
    n = pl.program_id(0)

    @pl.when(n == 0)
    def _init():
        counts_ref[...] = jnp.zeros_like(counts_ref)

    x = x_ref[...]
    xb = x.astype(jnp.bfloat16)
    isqr = isqr_ref[...]
    g_min = [jnp.full((NT, 1), jnp.inf, jnp.float32) for _ in GROUPS]
    g_idx = [jnp.zeros((NT, 1), jnp.int32) for _ in GROUPS]
    for c in range(NKC):
        eb = e_ref[c * KC:(c + 1) * KC, :].astype(jnp.bfloat16)
        mm = lax.dot_general(xb, eb, (((1,), (1,)), ((), ())),
                             preferred_element_type=jnp.float32)
        d = (csqr_ref[:, c * KC:(c + 1) * KC] + isqr) - 2.0 * mm
        ii = lax.broadcasted_iota(jnp.int32, (NT, KC), 1) + (c * KC)
        for g, (g0, g1) in enumerate(GROUPS):
            lo, hi = max(c * KC, g0), min((c + 1) * KC, g1)
            if lo >= hi:
                continue
            if lo == c * KC and hi == (c + 1) * KC:
                dg = d
            else:
                gmask = (ii >= g0) & (ii < g1)
                dg = jnp.where(gmask, d, jnp.inf)
            lm = jnp.min(dg, axis=1, keepdims=True)
            li = jnp.min(jnp.where(dg == lm, ii, K), axis=1, keepdims=True)
            upd = lm < g_min[g]
            g_min[g] = jnp.where(upd, lm, g_min[g])
            g_idx[g] = jnp.where(upd, li, g_idx[g])
    v = g_min[0].astype(jnp.bfloat16).astype(jnp.float32)
    run_idx = g_idx[0]
    for g in range(1, len(GROUPS)):
        upd = g_min[g] < v
        run_idx = jnp.where(upd, g_idx[g], run_idx)
        v = jnp.where(upd, g_min[g], v)
        v = v.astype(jnp.bfloat16).astype(jnp.float32)
    idx_ref[...] = run_idx

    hi_io = lax.broadcasted_iota(jnp.int32, (NT, 64), 1)
    lo_io = lax.broadcasted_iota(jnp.int32, (NT, 128), 1)
    oh_hi = jnp.where((run_idx >> 7) == hi_io, 1.0, 0.0).astype(jnp.bfloat16)
    oh_lo = jnp.where((run_idx & 127) == lo_io, 1.0, 0.0).astype(jnp.bfloat16)
    counts_ref[...] += lax.dot_general(
        oh_hi, oh_lo, (((0,), (0,)), ((), ())),
        preferred_element_type=jnp.float32)

    @pl.when(n == pl.num_programs(0) - 1)
    def _fin():
        p = counts_ref[...] * (1.0 / N)
        perp_ref[...] = jnp.exp(-jnp.sum(p * jnp.log(p + 1e-10))).reshape(1, 1)


def _dist_argmin(flat_x, emb, csqr, isqr):
    return pl.pallas_call(
        _dist_argmin_body,
        grid=(N // NT,),
        in_specs=[
            pl.BlockSpec((NT, D), lambda n: (n, 0)),
            pl.BlockSpec((K, D), lambda n: (0, 0)),
            pl.BlockSpec((1, K), lambda n: (0, 0)),
            pl.BlockSpec((NT, 1), lambda n: (n, 0)),
        ],
        out_specs=[
            pl.BlockSpec((NT, 1), lambda n: (n, 0)),
            pl.BlockSpec((1, 1), lambda n: (0, 0)),
        ],
        out_shape=[
            jax.ShapeDtypeStruct((N, 1), jnp.int32),
            jax.ShapeDtypeStruct((1, 1), jnp.float32),
        ],
        scratch_shapes=[
            pltpu.VMEM((64, 128), jnp.float32),
        ],
        compiler_params=pltpu.CompilerParams(
            dimension_semantics=("arbitrary",)),
    )(flat_x, emb, csqr, isqr)


def _sc_gather(table, idx):
    mesh = plsc.VectorSubcoreMesh(core_axis_name="c", subcore_axis_name="s")

    @functools.partial(
        pl.kernel, mesh=mesh,
        out_type=jax.ShapeDtypeStruct((N, D), jnp.float32),
        scratch_types=[
            pltpu.VMEM((ROWS_PER_W,), jnp.int32),
            pltpu.VMEM((GCH, D), jnp.float32),
            pltpu.VMEM((GCH, D), jnp.float32),
            pltpu.SemaphoreType.DMA,
            pltpu.SemaphoreType.DMA,
        ],
    )
    def gather_kernel(table_hbm, idx_hbm, out_hbm, idx_v, buf0, buf1, s0, s1):
        wid = lax.axis_index("s") * SC_NC + lax.axis_index("c")
        base = wid * ROWS_PER_W
        pltpu.sync_copy(idx_hbm.at[pl.ds(base, ROWS_PER_W)], idx_v)
        bufs, sems = (buf0, buf1), (s0, s1)
        handles = [None, None]
        handles[0] = pltpu.async_copy(
            table_hbm.at[idx_v.at[pl.ds(0, GCH)]], bufs[0], sems[0])
        for c in range(NGCH):
            if c + 1 < NGCH:
                nb = (c + 1) % 2
                handles[nb] = pltpu.async_copy(
                    table_hbm.at[idx_v.at[pl.ds((c + 1) * GCH, GCH)]],
                    bufs[nb], sems[nb])
            b = c % 2
            handles[b].wait()
            pltpu.sync_copy(bufs[b], out_hbm.at[pl.ds(base + c * GCH, GCH)])

    return gather_kernel(table, idx)


def kernel(inputs, embedding_weight):
    x = jnp.transpose(inputs, (0, 2, 3, 1))
    flat_x = x.reshape(N, D)
    csqr = jnp.sum(embedding_weight ** 2, axis=1)[None, :]
    isqr = jnp.sum(flat_x ** 2, axis=1, keepdims=True)
    idx2, perp = _dist_argmin(flat_x, embedding_weight, csqr, isqr)
    indices_out = idx2.reshape(B, H, W)
    q_flat = _sc_gather(embedding_weight, idx2.reshape(N))
    q = q_flat.reshape(B, H, W, D)
    q = x + lax.stop_gradient(q - x)
    quantized = jnp.transpose(q, (0, 3, 1, 2))
    return indices_out, quantized, perp.reshape(())
